# baseline (speedup 1.0000x reference)
import functools

import jax
import jax.numpy as jnp
from jax import lax
from jax.experimental import pallas as pl
from jax.experimental.pallas import tpu as pltpu, tpu_sc as plsc

NUM_CORES = 2
NUM_SUBCORES = 16
LANES = 16
NW = NUM_CORES * NUM_SUBCORES

BATCH = 16384
EMB_K = 16
NCLS = 5
BPW = BATCH // NW
CHUNK = 128
NCHUNK = BPW // CHUNK
GROUPS_PER_BLOCK = 4
ROWS_PER_BLOCK = GROUPS_PER_BLOCK * LANES
NBLOCKS = BPW // ROWS_PER_BLOCK


def _sc_body(W_hbm, H_hbm, uidx_hbm, vidx_hbm, wub_hbm, bb_hbm, out_hbm,
             uidx_v, vidx_v, urows, vrows, outv, wubv, bbv, sem):
    wid = lax.axis_index("s") * NUM_CORES + lax.axis_index("c")
    base = wid * BPW

    pltpu.sync_copy(uidx_hbm.at[pl.ds(wid * NCHUNK, NCHUNK)], uidx_v)
    pltpu.sync_copy(vidx_hbm.at[pl.ds(wid * NCHUNK, NCHUNK)], vidx_v)
    pltpu.sync_copy(wub_hbm, wubv)
    pltpu.sync_copy(bb_hbm, bbv)

    copies = []
    for c in range(NCHUNK):
        copies.append(pltpu.async_copy(
            W_hbm.at[uidx_v.at[c]], urows.at[pl.ds(c * CHUNK, CHUNK)], sem))
        copies.append(pltpu.async_copy(
            H_hbm.at[vidx_v.at[c]], vrows.at[pl.ds(c * CHUNK, CHUNK)], sem))
    for cp in copies:
        cp.wait()

    iota = lax.iota(jnp.int32, LANES)
    bvecs = [bbv[j] for j in range(NCLS)]

    def block(gb, carry):
        row0 = gb * ROWS_PER_BLOCK
        row_idx = [row0 + g * LANES + iota for g in range(GROUPS_PER_BLOCK)]
        acc = [[bvecs[j] for j in range(NCLS)]
               for _ in range(GROUPS_PER_BLOCK)]
        for k in range(2 * EMB_K):
            src = urows if k < EMB_K else vrows
            colv = jnp.full((LANES,), k % EMB_K, jnp.int32)
            wvecs = [wubv[j, k] for j in range(NCLS)]
            for g in range(GROUPS_PER_BLOCK):
                z = plsc.load_gather(src, [row_idx[g], colv])
                for j in range(NCLS):
                    acc[g][j] = acc[g][j] + z * wvecs[j]
        for g in range(GROUPS_PER_BLOCK):
            h = acc[g]
            m = h[0]
            for j in range(1, NCLS):
                m = jnp.maximum(m, h[j])
            e = [jnp.exp(h[j] - m) for j in range(NCLS)]
            s = e[0]
            for j in range(1, NCLS):
                s = s + e[j]
            r = jnp.full((LANES,), 1.0, jnp.float32) / s
            for j in range(NCLS):
                plsc.store_scatter(
                    outv, [row_idx[g], jnp.full((LANES,), j, jnp.int32)],
                    e[j] * r)
        return carry

    lax.fori_loop(0, NBLOCKS, block, 0)

    pltpu.sync_copy(outv, out_hbm.at[pl.ds(base, BPW)])


@functools.partial(jax.jit, static_argnames=())
def _run(W, H, uidx, vidx, wub, bb):
    mesh = plsc.VectorSubcoreMesh(
        core_axis_name="c", subcore_axis_name="s",
        num_cores=NUM_CORES, num_subcores=NUM_SUBCORES)
    return pl.kernel(
        _sc_body,
        out_type=jax.ShapeDtypeStruct((BATCH, NCLS), jnp.float32),
        mesh=mesh,
        compiler_params=pltpu.CompilerParams(
            needs_layout_passes=False, use_tc_tiling_on_sc=False),
        scratch_types=[
            pltpu.VMEM((NCHUNK, CHUNK), jnp.int32),
            pltpu.VMEM((NCHUNK, CHUNK), jnp.int32),
            pltpu.VMEM((BPW, EMB_K), jnp.float32),
            pltpu.VMEM((BPW, EMB_K), jnp.float32),
            pltpu.VMEM((BPW, NCLS), jnp.float32),
            pltpu.VMEM((NCLS, 2 * EMB_K, LANES), jnp.float32),
            pltpu.VMEM((NCLS, LANES), jnp.float32),
            pltpu.SemaphoreType.DMA,
        ],
    )(W, H, uidx, vidx, wub, bb)


def kernel(x, W, H, lin_w, lin_b):
    uidx = x[:, 0].reshape(NW * NCHUNK, CHUNK)
    vidx = x[:, 1].reshape(NW * NCHUNK, CHUNK)
    wub = jnp.broadcast_to(lin_w.reshape(NCLS, 2 * EMB_K, 1),
                           (NCLS, 2 * EMB_K, LANES))
    bb = jnp.broadcast_to(lin_b.reshape(NCLS, 1), (NCLS, LANES))
    return _run(W, H, uidx, vidx, wub, bb)
